# Initial kernel scaffold; baseline (speedup 1.0000x reference)
import jax
import jax.numpy as jnp
from jax.experimental import pallas as pl


def kernel(x, edge_index, W_l0, W_r0, att0, b0, W_l1, W_r1, att1, b1):
    raise NotImplementedError("write your pallas kernel here")


# speedup vs baseline: 32.8278x; 32.8278x over previous
import functools

import numpy as np

import jax
import jax.numpy as jnp
from jax import lax
from jax.experimental import pallas as pl
from jax.experimental.pallas import tpu as pltpu
from jax.experimental.pallas import tpu_sc as plsc

_N = 10000
_E = 160000
_D_IN = 256
_HID = 8
_HEADS = 8
_OUT = 40

_NACC = 10240
_DUMMY = 10000
_NW = 32
_CH = 128
_PER_W = 5376
_NCHUNK = _PER_W // _CH
_EW = _NW * _PER_W
_MBLK = 1280
_RPS = _NACC // 16

_f32 = jnp.float32
_i32 = jnp.int32


def _k1_body(x_ref, wl_ref, wr_ref, att_ref, sel_ref, xl_ref, xr_ref, m_ref):
    xl = jnp.dot(x_ref[...], wl_ref[...], preferred_element_type=_f32)
    xr = jnp.dot(x_ref[...], wr_ref[...], preferred_element_type=_f32)
    xl_ref[...] = xl
    xr_ref[...] = xr
    z = xl + xr
    t = (jnp.maximum(z, 0.0) + 0.2 * jnp.minimum(z, 0.0)) * att_ref[...]
    m_ref[...] = jnp.dot(t, sel_ref[...], preferred_element_type=_f32)


def _k1(xpad, wl, wr, att_row, sel):
    nco = wl.shape[1]
    nm = sel.shape[1]
    return pl.pallas_call(
        _k1_body,
        grid=(_NACC // _MBLK,),
        in_specs=[
            pl.BlockSpec((_MBLK, _D_IN), lambda i: (i, 0)),
            pl.BlockSpec((_D_IN, nco), lambda i: (0, 0)),
            pl.BlockSpec((_D_IN, nco), lambda i: (0, 0)),
            pl.BlockSpec((1, nco), lambda i: (0, 0)),
            pl.BlockSpec((nco, nm), lambda i: (0, 0)),
        ],
        out_specs=[
            pl.BlockSpec((_MBLK, nco), lambda i: (i, 0)),
            pl.BlockSpec((_MBLK, nco), lambda i: (i, 0)),
            pl.BlockSpec((_MBLK, nm), lambda i: (i, 0)),
        ],
        out_shape=[
            jax.ShapeDtypeStruct((_NACC, nco), _f32),
            jax.ShapeDtypeStruct((_NACC, nco), _f32),
            jax.ShapeDtypeStruct((_NACC, nm), _f32),
        ],
    )(xpad, wl, wr, att_row, sel)


def _edge_pass0(xl_hbm, xr_hbm, m_hbm, src_hbm, dst_hbm, att_hbm, out_hbm,
                acc_sh, sidx, didx, gl, gr, mg, msg, sfold, attv,
                sem0, sem1, sem2):
    cid = lax.axis_index("c")
    sid = lax.axis_index("s")
    wid = sid * 2 + cid

    pltpu.sync_copy(att_hbm, attv)

    lane = lax.iota(_i32, 16)
    idx_a = lane % 8
    idx_b = idx_a + 8
    mask10 = jnp.where(lane < 8, 1.0, 0.0).astype(_f32)
    zero16 = jnp.zeros((16,), _f32)

    def _zb(j, c):
        for k in range(5):
            msg[j, pl.ds(16 * k, 16)] = zero16
        return c
    lax.fori_loop(0, _CH, _zb, 0)
    for i in range(5):
        pltpu.sync_copy(msg, acc_sh.at[pl.ds(sid * _RPS + i * _CH, _CH)])
    plsc.subcore_barrier()

    def _chunk(ci, c):
        base = wid * _PER_W + ci * _CH
        pltpu.sync_copy(src_hbm.at[pl.ds(base, _CH)], sidx)
        pltpu.sync_copy(dst_hbm.at[pl.ds(base, _CH)], didx)
        c1 = pltpu.async_copy(xl_hbm.at[sidx], gl, sem0)
        c2 = pltpu.async_copy(xr_hbm.at[didx], gr, sem1)
        c3 = pltpu.async_copy(m_hbm.at[didx], mg, sem2)
        c1.wait()
        c2.wait()
        c3.wait()

        def _edge(j, cc):
            jb = j * 16
            tsum = zero16
            for k in range(4):
                a = gl[j, pl.ds(16 * k, 16)]
                b = gr[j, pl.ds(16 * k, 16)]
                z = a + b
                t = (jnp.maximum(z, 0.0) + 0.2 * jnp.minimum(z, 0.0)) \
                    * attv[pl.ds(16 * k, 16)]
                tsum = tsum + t
            sfold[pl.ds(jb, 16)] = tsum
            alo = plsc.load_gather(sfold, [jb + idx_a])
            ahi = plsc.load_gather(sfold, [jb + idx_b])
            exb = jnp.exp((alo + ahi) - mg[j, :])
            for k in range(4):
                msg[j, pl.ds(16 * k, 16)] = gl[j, pl.ds(16 * k, 16)] * exb
            msg[j, pl.ds(64, 16)] = exb * mask10
            return cc
        lax.fori_loop(0, _CH, _edge, 0)
        pltpu.sync_copy(msg, acc_sh.at[didx], add=True)
        return c
    lax.fori_loop(0, _NCHUNK, _chunk, 0)
    plsc.subcore_barrier()
    pltpu.sync_copy(acc_sh.at[pl.ds(sid * _RPS, _RPS)],
                    out_hbm.at[cid, pl.ds(sid * _RPS, _RPS)])


def _k2(xl, xr, m0, srcp, dstp, att):
    mesh = plsc.VectorSubcoreMesh(core_axis_name="c", subcore_axis_name="s")
    f = pl.kernel(
        _edge_pass0,
        out_type=jax.ShapeDtypeStruct((2, _NACC, 80), _f32),
        mesh=mesh,
        compiler_params=pltpu.CompilerParams(needs_layout_passes=False, use_tc_tiling_on_sc=False),
        scratch_types=[
            pltpu.VMEM_SHARED((_NACC, 80), _f32),
            pltpu.VMEM((_CH,), _i32),
            pltpu.VMEM((_CH,), _i32),
            pltpu.VMEM((_CH, 64), _f32),
            pltpu.VMEM((_CH, 64), _f32),
            pltpu.VMEM((_CH, 16), _f32),
            pltpu.VMEM((_CH, 80), _f32),
            pltpu.VMEM((_CH * 16,), _f32),
            pltpu.VMEM((64,), _f32),
            pltpu.SemaphoreType.DMA,
            pltpu.SemaphoreType.DMA,
            pltpu.SemaphoreType.DMA,
        ],
    )
    return f(xl, xr, m0, srcp, dstp, att)


def _edge_pass1(xl_hbm, xr_hbm, m_hbm, src_hbm, dst_hbm, att_hbm, out_hbm,
                acc_sh, sidx, didx, gl, gr, mg, msg, attv,
                sem0, sem1, sem2):
    cid = lax.axis_index("c")
    sid = lax.axis_index("s")
    wid = sid * 2 + cid

    pltpu.sync_copy(att_hbm, attv)

    lane = lax.iota(_i32, 16)
    unit8 = jnp.where(lane == 8, 1.0, 0.0).astype(_f32)
    zero16 = jnp.zeros((16,), _f32)

    def _zb(j, c):
        for k in range(3):
            msg[j, pl.ds(16 * k, 16)] = zero16
        return c
    lax.fori_loop(0, _CH, _zb, 0)
    for i in range(5):
        pltpu.sync_copy(msg, acc_sh.at[pl.ds(sid * _RPS + i * _CH, _CH)])
    plsc.subcore_barrier()

    def _chunk(ci, c):
        base = wid * _PER_W + ci * _CH
        pltpu.sync_copy(src_hbm.at[pl.ds(base, _CH)], sidx)
        pltpu.sync_copy(dst_hbm.at[pl.ds(base, _CH)], didx)
        c1 = pltpu.async_copy(xl_hbm.at[sidx], gl, sem0)
        c2 = pltpu.async_copy(xr_hbm.at[didx], gr, sem1)
        c3 = pltpu.async_copy(m_hbm.at[didx], mg, sem2)
        c1.wait()
        c2.wait()
        c3.wait()

        def _edge(j, cc):
            tsum = zero16
            for k in range(3):
                a = gl[j, pl.ds(16 * k, 16)]
                b = gr[j, pl.ds(16 * k, 16)]
                z = a + b
                t = (jnp.maximum(z, 0.0) + 0.2 * jnp.minimum(z, 0.0)) \
                    * attv[pl.ds(16 * k, 16)]
                tsum = tsum + t
            alpha = jnp.sum(tsum)
            exv = jnp.exp(jnp.full((16,), alpha, _f32) - mg[j, :])
            msg[j, pl.ds(0, 16)] = gl[j, pl.ds(0, 16)] * exv
            msg[j, pl.ds(16, 16)] = gl[j, pl.ds(16, 16)] * exv
            msg[j, pl.ds(32, 16)] = gl[j, pl.ds(32, 16)] * exv \
                + exv * unit8
            return cc
        lax.fori_loop(0, _CH, _edge, 0)
        pltpu.sync_copy(msg, acc_sh.at[didx], add=True)
        return c
    lax.fori_loop(0, _NCHUNK, _chunk, 0)
    plsc.subcore_barrier()
    pltpu.sync_copy(acc_sh.at[pl.ds(sid * _RPS, _RPS)],
                    out_hbm.at[cid, pl.ds(sid * _RPS, _RPS)])


def _k4(xl, xr, m1, srcp, dstp, att):
    mesh = plsc.VectorSubcoreMesh(core_axis_name="c", subcore_axis_name="s")
    f = pl.kernel(
        _edge_pass1,
        out_type=jax.ShapeDtypeStruct((2, _NACC, 48), _f32),
        mesh=mesh,
        compiler_params=pltpu.CompilerParams(needs_layout_passes=False, use_tc_tiling_on_sc=False),
        scratch_types=[
            pltpu.VMEM_SHARED((_NACC, 48), _f32),
            pltpu.VMEM((_CH,), _i32),
            pltpu.VMEM((_CH,), _i32),
            pltpu.VMEM((_CH, 48), _f32),
            pltpu.VMEM((_CH, 48), _f32),
            pltpu.VMEM((_CH, 16), _f32),
            pltpu.VMEM((_CH, 48), _f32),
            pltpu.VMEM((48,), _f32),
            pltpu.SemaphoreType.DMA,
            pltpu.SemaphoreType.DMA,
            pltpu.SemaphoreType.DMA,
        ],
    )
    return f(xl, xr, m1, srcp, dstp, att)


def _k3_body(acc_ref, b0_ref, wl_ref, wr_ref, att_ref, sel_ref,
             xl_ref, xr_ref, m_ref):
    a = acc_ref[0] + acc_ref[1]
    den = a[:, 64:72]
    den8 = jnp.concatenate([den] * 8, axis=1)
    h = a[:, 0:64] / (den8 + 1e-16) + b0_ref[...]
    h = jnp.where(h > 0.0, h, jnp.exp(jnp.minimum(h, 0.0)) - 1.0)
    xl = jnp.dot(h, wl_ref[...], preferred_element_type=_f32)
    xr = jnp.dot(h, wr_ref[...], preferred_element_type=_f32)
    xl_ref[...] = xl
    xr_ref[...] = xr
    z = xl + xr
    t = (jnp.maximum(z, 0.0) + 0.2 * jnp.minimum(z, 0.0)) * att_ref[...]
    m_ref[...] = jnp.dot(t, sel_ref[...], preferred_element_type=_f32)


def _k3(acc0, b0_row, wl1, wr1, att1_row, sel1):
    return pl.pallas_call(
        _k3_body,
        grid=(_NACC // _MBLK,),
        in_specs=[
            pl.BlockSpec((2, _MBLK, 80), lambda i: (0, i, 0)),
            pl.BlockSpec((1, 64), lambda i: (0, 0)),
            pl.BlockSpec((64, 48), lambda i: (0, 0)),
            pl.BlockSpec((64, 48), lambda i: (0, 0)),
            pl.BlockSpec((1, 48), lambda i: (0, 0)),
            pl.BlockSpec((48, 16), lambda i: (0, 0)),
        ],
        out_specs=[
            pl.BlockSpec((_MBLK, 48), lambda i: (i, 0)),
            pl.BlockSpec((_MBLK, 48), lambda i: (i, 0)),
            pl.BlockSpec((_MBLK, 16), lambda i: (i, 0)),
        ],
        out_shape=[
            jax.ShapeDtypeStruct((_NACC, 48), _f32),
            jax.ShapeDtypeStruct((_NACC, 48), _f32),
            jax.ShapeDtypeStruct((_NACC, 16), _f32),
        ],
    )(acc0, b0_row, wl1, wr1, att1_row, sel1)


def _k5_body(acc_ref, b1_ref, o_ref):
    a = acc_ref[0] + acc_ref[1]
    h2 = a[:, 0:40] / (a[:, 40:41] + 1e-16) + b1_ref[...]
    mx = jnp.max(h2, axis=1, keepdims=True)
    e = jnp.exp(h2 - mx)
    o_ref[...] = e / jnp.sum(e, axis=1, keepdims=True)


def _k5(acc1, b1_row):
    blk = 1000
    return pl.pallas_call(
        _k5_body,
        grid=(_N // blk,),
        in_specs=[
            pl.BlockSpec((2, blk, 48), lambda i: (0, i, 0)),
            pl.BlockSpec((1, 40), lambda i: (0, 0)),
        ],
        out_specs=pl.BlockSpec((blk, 40), lambda i: (i, 0)),
        out_shape=jax.ShapeDtypeStruct((_N, 40), _f32),
    )(acc1, b1_row)


def kernel(x, edge_index, W_l0, W_r0, att0, b0, W_l1, W_r1, att1, b1):
    p = np.arange(64)
    perm = (p % 8) * 8 + p // 8
    sel0 = np.zeros((64, 16), np.float32)
    sel0[p, p % 8] = 1.0
    sel0[p, p % 8 + 8] = 1.0
    sel1 = np.ones((48, 16), np.float32)

    wl0p = W_l0[:, perm]
    wr0p = W_r0[:, perm]
    att_cm = att0.reshape(64)[perm]
    b0_cm = b0[perm]

    xpad = jnp.pad(x, ((0, _NACC - _N), (0, 0)))
    xl0, xr0, m0 = _k1(xpad, wl0p, wr0p, att_cm[None], jnp.asarray(sel0))

    loop = jnp.arange(_N, dtype=_i32)
    padlen = _EW - (_E + _N)
    srcp = jnp.concatenate([edge_index[0].astype(_i32), loop,
                            jnp.full((padlen,), _DUMMY, _i32)])
    dstp = jnp.concatenate([edge_index[1].astype(_i32), loop,
                            jnp.full((padlen,), _DUMMY, _i32)])

    acc0 = _k2(xl0, xr0, m0, srcp, dstp, att_cm)

    wl1p = jnp.pad(W_l1[perm, :], ((0, 0), (0, 8)))
    wr1p = jnp.pad(W_r1[perm, :], ((0, 0), (0, 8)))
    att1p = jnp.pad(att1[0], (0, 8))
    xl1, xr1, m1 = _k3(acc0, b0_cm[None], wl1p, wr1p, att1p[None],
                       jnp.asarray(sel1))

    acc1 = _k4(xl1, xr1, m1, srcp, dstp, att1p)
    return _k5(acc1, b1[None])


# speedup vs baseline: 75.9089x; 2.3123x over previous
import functools

import numpy as np

import jax
import jax.numpy as jnp
from jax import lax
from jax.experimental import pallas as pl
from jax.experimental.pallas import tpu as pltpu
from jax.experimental.pallas import tpu_sc as plsc

_N = 10000
_E = 160000
_D_IN = 256
_HID = 8
_HEADS = 8
_OUT = 40

_NACC = 10240
_DUMMY = 10000
_NW = 32
_CH = 128
_PER_W = 5376
_NCHUNK = _PER_W // _CH
_EW = _NW * _PER_W
_MBLK = 1280
_RPS = _NACC // 16

_f32 = jnp.float32
_i32 = jnp.int32


def _k1_body(x_ref, wl_ref, wr_ref, att_ref, sel_ref, xl_ref, xr_ref, m_ref):
    xl = jnp.dot(x_ref[...], wl_ref[...], preferred_element_type=_f32)
    xr = jnp.dot(x_ref[...], wr_ref[...], preferred_element_type=_f32)
    xl_ref[...] = xl
    xr_ref[...] = xr
    z = xl + xr
    t = (jnp.maximum(z, 0.0) + 0.2 * jnp.minimum(z, 0.0)) * att_ref[...]
    m_ref[...] = jnp.dot(t, sel_ref[...], preferred_element_type=_f32)


def _k1(xpad, wl, wr, att_row, sel):
    nco = wl.shape[1]
    nm = sel.shape[1]
    return pl.pallas_call(
        _k1_body,
        grid=(_NACC // _MBLK,),
        in_specs=[
            pl.BlockSpec((_MBLK, _D_IN), lambda i: (i, 0)),
            pl.BlockSpec((_D_IN, nco), lambda i: (0, 0)),
            pl.BlockSpec((_D_IN, nco), lambda i: (0, 0)),
            pl.BlockSpec((1, nco), lambda i: (0, 0)),
            pl.BlockSpec((nco, nm), lambda i: (0, 0)),
        ],
        out_specs=[
            pl.BlockSpec((_MBLK, nco), lambda i: (i, 0)),
            pl.BlockSpec((_MBLK, nco), lambda i: (i, 0)),
            pl.BlockSpec((_MBLK, nm), lambda i: (i, 0)),
        ],
        out_shape=[
            jax.ShapeDtypeStruct((_NACC, nco), _f32),
            jax.ShapeDtypeStruct((_NACC, nco), _f32),
            jax.ShapeDtypeStruct((_NACC, nm), _f32),
        ],
    )(xpad, wl, wr, att_row, sel)


def _make_dma_helpers(ebase, src_hbm, dst_hbm, xl_hbm, xr_hbm, m_hbm,
                      acc_sh, sidx, didx, gl, gr, mg, msg, isem, gsem, ssem):
    def idx_issue(c, i3):
        pltpu.async_copy(src_hbm.at[pl.ds(ebase + c * _CH, _CH)], sidx[i3],
                         isem[i3])
        pltpu.async_copy(dst_hbm.at[pl.ds(ebase + c * _CH, _CH)], didx[i3],
                         isem[i3])

    def idx_wait(c, i3):
        pltpu.make_async_copy(src_hbm.at[pl.ds(ebase + c * _CH, _CH)],
                              sidx[i3], isem[i3]).wait()
        pltpu.make_async_copy(dst_hbm.at[pl.ds(ebase + c * _CH, _CH)],
                              didx[i3], isem[i3]).wait()

    def gat_issue(i2, i3):
        pltpu.async_copy(xl_hbm.at[sidx[i3]], gl[i2], gsem[i2])
        pltpu.async_copy(xr_hbm.at[didx[i3]], gr[i2], gsem[i2])
        pltpu.async_copy(m_hbm.at[didx[i3]], mg[i2], gsem[i2])

    def gat_wait(i2, i3):
        pltpu.make_async_copy(xl_hbm.at[sidx[i3]], gl[i2], gsem[i2]).wait()
        pltpu.make_async_copy(xr_hbm.at[didx[i3]], gr[i2], gsem[i2]).wait()
        pltpu.make_async_copy(m_hbm.at[didx[i3]], mg[i2], gsem[i2]).wait()

    def sc_issue(m2, i3):
        pltpu.async_copy(msg[m2], acc_sh.at[didx[i3]], ssem[m2], add=True)

    def sc_wait(m2, i3):
        pltpu.make_async_copy(msg[m2], acc_sh.at[didx[i3]], ssem[m2]).wait()

    return idx_issue, idx_wait, gat_issue, gat_wait, sc_issue, sc_wait


def _edge_pass0(xl_hbm, xr_hbm, m_hbm, src_hbm, dst_hbm, att_hbm, out_hbm,
                acc_sh, *s):
    sidx = s[0:3]
    didx = s[3:6]
    gl = s[6:8]
    gr = s[8:10]
    mg = s[10:12]
    msg = s[12:14]
    sfold = s[14]
    attv = s[15]
    isem = s[16:19]
    gsem = s[19:21]
    ssem = s[21:23]

    cid = lax.axis_index("c")
    sid = lax.axis_index("s")
    wid = sid * 2 + cid
    ebase = wid * _PER_W

    pltpu.sync_copy(att_hbm, attv)

    lane = lax.iota(_i32, 16)
    idx_a = lane % 8
    idx_b = idx_a + 8
    zero16 = jnp.zeros((16,), _f32)

    def _zb(j, c):
        for k in range(5):
            msg[0][j, pl.ds(16 * k, 16)] = zero16
        return c
    lax.fori_loop(0, _CH, _zb, 0)
    for i in range(5):
        pltpu.sync_copy(msg[0], acc_sh.at[pl.ds(sid * _RPS + i * _CH, _CH)])
    plsc.subcore_barrier()

    idx_issue, idx_wait, gat_issue, gat_wait, sc_issue, sc_wait = \
        _make_dma_helpers(ebase, src_hbm, dst_hbm, xl_hbm, xr_hbm, m_hbm,
                          acc_sh, sidx, didx, gl, gr, mg, msg,
                          isem, gsem, ssem)

    idx_issue(0, 0)
    idx_wait(0, 0)
    gat_issue(0, 0)
    idx_issue(1, 1)

    a0 = attv[pl.ds(0, 16)]
    a1 = attv[pl.ds(16, 16)]
    a2 = attv[pl.ds(32, 16)]
    a3 = attv[pl.ds(48, 16)]

    def _group(g, cval):
        for b in range(6):
            ci = g * 6 + b
            i3, i2 = b % 3, b % 2

            @pl.when(ci >= 1)
            def _():
                sc_wait((b + 1) % 2, (b + 2) % 3)

            @pl.when(ci <= _NCHUNK - 3)
            def _():
                idx_issue(ci + 2, (b + 2) % 3)

            @pl.when(ci <= _NCHUNK - 2)
            def _():
                idx_wait(ci + 1, (b + 1) % 3)
                gat_issue((b + 1) % 2, (b + 1) % 3)

            gat_wait(i2, i3)
            glb, grb, mgb, msgb = gl[i2], gr[i2], mg[i2], msg[i2]

            def _edge(j, atts):
                aa0, aa1, aa2, aa3 = atts
                jb = j * 16
                z0 = glb[j, pl.ds(0, 16)] + grb[j, pl.ds(0, 16)]
                z1 = glb[j, pl.ds(16, 16)] + grb[j, pl.ds(16, 16)]
                z2 = glb[j, pl.ds(32, 16)] + grb[j, pl.ds(32, 16)]
                z3 = glb[j, pl.ds(48, 16)] + grb[j, pl.ds(48, 16)]
                t = (jnp.maximum(z0, 0.0) + 0.2 * jnp.minimum(z0, 0.0)) * aa0
                t = t + (jnp.maximum(z1, 0.0)
                         + 0.2 * jnp.minimum(z1, 0.0)) * aa1
                t = t + (jnp.maximum(z2, 0.0)
                         + 0.2 * jnp.minimum(z2, 0.0)) * aa2
                t = t + (jnp.maximum(z3, 0.0)
                         + 0.2 * jnp.minimum(z3, 0.0)) * aa3
                sfold[pl.ds(jb, 16)] = t
                alo = plsc.load_gather(sfold, [jb + idx_a])
                ahi = plsc.load_gather(sfold, [jb + idx_b])
                exb = jnp.exp((alo + ahi) - mgb[j, :])
                msgb[j, pl.ds(0, 16)] = glb[j, pl.ds(0, 16)] * exb
                msgb[j, pl.ds(16, 16)] = glb[j, pl.ds(16, 16)] * exb
                msgb[j, pl.ds(32, 16)] = glb[j, pl.ds(32, 16)] * exb
                msgb[j, pl.ds(48, 16)] = glb[j, pl.ds(48, 16)] * exb
                msgb[j, pl.ds(64, 16)] = exb
                return atts

            plsc.parallel_loop(0, _CH, unroll=2, carry=(a0, a1, a2, a3))(_edge)
            sc_issue(i2, i3)
        return cval

    lax.fori_loop(0, _NCHUNK // 6, _group, 0)
    sc_wait((_NCHUNK - 1) % 2, (_NCHUNK - 1) % 3)
    plsc.subcore_barrier()
    pltpu.sync_copy(acc_sh.at[pl.ds(sid * _RPS, _RPS)],
                    out_hbm.at[cid, pl.ds(sid * _RPS, _RPS)])


def _k2(xl, xr, m0, srcp, dstp, att):
    mesh = plsc.VectorSubcoreMesh(core_axis_name="c", subcore_axis_name="s")
    f = pl.kernel(
        _edge_pass0,
        out_type=jax.ShapeDtypeStruct((2, _NACC, 80), _f32),
        mesh=mesh,
        compiler_params=pltpu.CompilerParams(
            needs_layout_passes=False, use_tc_tiling_on_sc=False),
        scratch_types=(
            [pltpu.VMEM_SHARED((_NACC, 80), _f32)]
            + [pltpu.VMEM((_CH,), _i32)] * 6
            + [pltpu.VMEM((_CH, 64), _f32)] * 4
            + [pltpu.VMEM((_CH, 16), _f32)] * 2
            + [pltpu.VMEM((_CH, 80), _f32)] * 2
            + [pltpu.VMEM((_CH * 16,), _f32)]
            + [pltpu.VMEM((64,), _f32)]
            + [pltpu.SemaphoreType.DMA] * 7
        ),
    )
    return f(xl, xr, m0, srcp, dstp, att)


def _edge_pass1(xl_hbm, xr_hbm, m_hbm, src_hbm, dst_hbm, att_hbm, out_hbm,
                acc_sh, *s):
    sidx = s[0:3]
    didx = s[3:6]
    gl = s[6:8]
    gr = s[8:10]
    mg = s[10:12]
    msg = s[12:14]
    attv = s[14]
    isem = s[15:18]
    gsem = s[18:20]
    ssem = s[20:22]

    cid = lax.axis_index("c")
    sid = lax.axis_index("s")
    wid = sid * 2 + cid
    ebase = wid * _PER_W

    pltpu.sync_copy(att_hbm, attv)

    lane = lax.iota(_i32, 16)
    unit8 = jnp.where(lane == 8, 1.0, 0.0).astype(_f32)
    zero16 = jnp.zeros((16,), _f32)

    def _zb(j, c):
        for k in range(3):
            msg[0][j, pl.ds(16 * k, 16)] = zero16
        return c
    lax.fori_loop(0, _CH, _zb, 0)
    for i in range(5):
        pltpu.sync_copy(msg[0], acc_sh.at[pl.ds(sid * _RPS + i * _CH, _CH)])
    plsc.subcore_barrier()

    idx_issue, idx_wait, gat_issue, gat_wait, sc_issue, sc_wait = \
        _make_dma_helpers(ebase, src_hbm, dst_hbm, xl_hbm, xr_hbm, m_hbm,
                          acc_sh, sidx, didx, gl, gr, mg, msg,
                          isem, gsem, ssem)

    idx_issue(0, 0)
    idx_wait(0, 0)
    gat_issue(0, 0)
    idx_issue(1, 1)

    a0 = attv[pl.ds(0, 16)]
    a1 = attv[pl.ds(16, 16)]
    a2 = attv[pl.ds(32, 16)]

    def _group(g, cval):
        for b in range(6):
            ci = g * 6 + b
            i3, i2 = b % 3, b % 2

            @pl.when(ci >= 1)
            def _():
                sc_wait((b + 1) % 2, (b + 2) % 3)

            @pl.when(ci <= _NCHUNK - 3)
            def _():
                idx_issue(ci + 2, (b + 2) % 3)

            @pl.when(ci <= _NCHUNK - 2)
            def _():
                idx_wait(ci + 1, (b + 1) % 3)
                gat_issue((b + 1) % 2, (b + 1) % 3)

            gat_wait(i2, i3)
            glb, grb, mgb, msgb = gl[i2], gr[i2], mg[i2], msg[i2]

            def _edge(j, atts):
                aa0, aa1, aa2 = atts
                z0 = glb[j, pl.ds(0, 16)] + grb[j, pl.ds(0, 16)]
                z1 = glb[j, pl.ds(16, 16)] + grb[j, pl.ds(16, 16)]
                z2 = glb[j, pl.ds(32, 16)] + grb[j, pl.ds(32, 16)]
                t = (jnp.maximum(z0, 0.0) + 0.2 * jnp.minimum(z0, 0.0)) * aa0
                t = t + (jnp.maximum(z1, 0.0)
                         + 0.2 * jnp.minimum(z1, 0.0)) * aa1
                t = t + (jnp.maximum(z2, 0.0)
                         + 0.2 * jnp.minimum(z2, 0.0)) * aa2
                alpha = jnp.sum(t)
                exv = jnp.exp(jnp.full((16,), alpha, _f32) - mgb[j, :])
                msgb[j, pl.ds(0, 16)] = glb[j, pl.ds(0, 16)] * exv
                msgb[j, pl.ds(16, 16)] = glb[j, pl.ds(16, 16)] * exv
                msgb[j, pl.ds(32, 16)] = glb[j, pl.ds(32, 16)] * exv \
                    + exv * unit8
                return atts

            plsc.parallel_loop(0, _CH, unroll=2, carry=(a0, a1, a2))(_edge)
            sc_issue(i2, i3)
        return cval

    lax.fori_loop(0, _NCHUNK // 6, _group, 0)
    sc_wait((_NCHUNK - 1) % 2, (_NCHUNK - 1) % 3)
    plsc.subcore_barrier()
    pltpu.sync_copy(acc_sh.at[pl.ds(sid * _RPS, _RPS)],
                    out_hbm.at[cid, pl.ds(sid * _RPS, _RPS)])


def _k4(xl, xr, m1, srcp, dstp, att):
    mesh = plsc.VectorSubcoreMesh(core_axis_name="c", subcore_axis_name="s")
    f = pl.kernel(
        _edge_pass1,
        out_type=jax.ShapeDtypeStruct((2, _NACC, 48), _f32),
        mesh=mesh,
        compiler_params=pltpu.CompilerParams(
            needs_layout_passes=False, use_tc_tiling_on_sc=False),
        scratch_types=(
            [pltpu.VMEM_SHARED((_NACC, 48), _f32)]
            + [pltpu.VMEM((_CH,), _i32)] * 6
            + [pltpu.VMEM((_CH, 48), _f32)] * 4
            + [pltpu.VMEM((_CH, 16), _f32)] * 2
            + [pltpu.VMEM((_CH, 48), _f32)] * 2
            + [pltpu.VMEM((48,), _f32)]
            + [pltpu.SemaphoreType.DMA] * 7
        ),
    )
    return f(xl, xr, m1, srcp, dstp, att)


def _k3_body(acc_ref, b0_ref, wl_ref, wr_ref, att_ref, sel_ref,
             xl_ref, xr_ref, m_ref):
    a = acc_ref[0] + acc_ref[1]
    den = a[:, 64:72]
    den8 = jnp.concatenate([den] * 8, axis=1)
    h = a[:, 0:64] / (den8 + 1e-16) + b0_ref[...]
    h = jnp.where(h > 0.0, h, jnp.exp(jnp.minimum(h, 0.0)) - 1.0)
    xl = jnp.dot(h, wl_ref[...], preferred_element_type=_f32)
    xr = jnp.dot(h, wr_ref[...], preferred_element_type=_f32)
    xl_ref[...] = xl
    xr_ref[...] = xr
    z = xl + xr
    t = (jnp.maximum(z, 0.0) + 0.2 * jnp.minimum(z, 0.0)) * att_ref[...]
    m_ref[...] = jnp.dot(t, sel_ref[...], preferred_element_type=_f32)


def _k3(acc0, b0_row, wl1, wr1, att1_row, sel1):
    return pl.pallas_call(
        _k3_body,
        grid=(_NACC // _MBLK,),
        in_specs=[
            pl.BlockSpec((2, _MBLK, 80), lambda i: (0, i, 0)),
            pl.BlockSpec((1, 64), lambda i: (0, 0)),
            pl.BlockSpec((64, 48), lambda i: (0, 0)),
            pl.BlockSpec((64, 48), lambda i: (0, 0)),
            pl.BlockSpec((1, 48), lambda i: (0, 0)),
            pl.BlockSpec((48, 16), lambda i: (0, 0)),
        ],
        out_specs=[
            pl.BlockSpec((_MBLK, 48), lambda i: (i, 0)),
            pl.BlockSpec((_MBLK, 48), lambda i: (i, 0)),
            pl.BlockSpec((_MBLK, 16), lambda i: (i, 0)),
        ],
        out_shape=[
            jax.ShapeDtypeStruct((_NACC, 48), _f32),
            jax.ShapeDtypeStruct((_NACC, 48), _f32),
            jax.ShapeDtypeStruct((_NACC, 16), _f32),
        ],
    )(acc0, b0_row, wl1, wr1, att1_row, sel1)


def _k5_body(acc_ref, b1_ref, o_ref):
    a = acc_ref[0] + acc_ref[1]
    h2 = a[:, 0:40] / (a[:, 40:41] + 1e-16) + b1_ref[...]
    mx = jnp.max(h2, axis=1, keepdims=True)
    e = jnp.exp(h2 - mx)
    o_ref[...] = e / jnp.sum(e, axis=1, keepdims=True)


def _k5(acc1, b1_row):
    blk = 1000
    return pl.pallas_call(
        _k5_body,
        grid=(_N // blk,),
        in_specs=[
            pl.BlockSpec((2, blk, 48), lambda i: (0, i, 0)),
            pl.BlockSpec((1, 40), lambda i: (0, 0)),
        ],
        out_specs=pl.BlockSpec((blk, 40), lambda i: (i, 0)),
        out_shape=jax.ShapeDtypeStruct((_N, 40), _f32),
    )(acc1, b1_row)


def kernel(x, edge_index, W_l0, W_r0, att0, b0, W_l1, W_r1, att1, b1):
    p = np.arange(64)
    perm = (p % 8) * 8 + p // 8
    sel0 = np.zeros((64, 16), np.float32)
    sel0[p, p % 8] = 1.0
    sel0[p, p % 8 + 8] = 1.0
    sel1 = np.ones((48, 16), np.float32)

    wl0p = W_l0[:, perm]
    wr0p = W_r0[:, perm]
    att_cm = att0.reshape(64)[perm]
    b0_cm = b0[perm]

    xpad = jnp.pad(x, ((0, _NACC - _N), (0, 0)))
    xl0, xr0, m0 = _k1(xpad, wl0p, wr0p, att_cm[None], jnp.asarray(sel0))

    loop = jnp.arange(_N, dtype=_i32)
    padlen = _EW - (_E + _N)
    srcp = jnp.concatenate([edge_index[0].astype(_i32), loop,
                            jnp.full((padlen,), _DUMMY, _i32)])
    dstp = jnp.concatenate([edge_index[1].astype(_i32), loop,
                            jnp.full((padlen,), _DUMMY, _i32)])

    acc0 = _k2(xl0, xr0, m0, srcp, dstp, att_cm)

    wl1p = jnp.pad(W_l1[perm, :], ((0, 0), (0, 8)))
    wr1p = jnp.pad(W_r1[perm, :], ((0, 0), (0, 8)))
    att1p = jnp.pad(att1[0], (0, 8))
    xl1, xr1, m1 = _k3(acc0, b0_cm[None], wl1p, wr1p, att1p[None],
                       jnp.asarray(sel1))

    acc1 = _k4(xl1, xr1, m1, srcp, dstp, att1p)
    return _k5(acc1, b1[None])
